# baseline (speedup 1.0000x reference)
import functools

import jax
import jax.numpy as jnp
from jax import lax
from jax.experimental import pallas as pl
from jax.experimental.pallas import tpu as pltpu
from jax.experimental.pallas import tpu_sc as plsc

_ZERO_L = (1, 3, 4)


def _sc_gather(p4, rp_flat, zeros_blk, B, T, L, M, N, TT):
    NC, NS = 2, 16
    NW = NC * NS
    b_per_w = B // NW
    live_l = tuple(l for l in range(L) if l not in _ZERO_L)

    mesh = plsc.VectorSubcoreMesh(core_axis_name="c", subcore_axis_name="s")

    @functools.partial(
        pl.kernel,
        mesh=mesh,
        compiler_params=pltpu.CompilerParams(use_tc_tiling_on_sc=False),
        out_type=jax.ShapeDtypeStruct((B, N, TT, L, 128), jnp.float32),
        scratch_types=[
            pltpu.VMEM((2, N, TT, L, 128), jnp.float32),
            pltpu.VMEM((B * L + 8,), jnp.int32),
            pltpu.SemaphoreType.DMA,
            pltpu.SemaphoreType.DMA,
        ],
    )
    def k(p_hbm, rp_hbm, z_hbm, out_hbm, buf, rp_v, sem, wsem):
        wid = lax.axis_index("s") * NC + lax.axis_index("c")
        pltpu.sync_copy(rp_hbm, rp_v.at[pl.ds(0, B * L)])
        pltpu.sync_copy(z_hbm, buf.at[0])
        pltpu.sync_copy(z_hbm, buf.at[1])

        b0 = wid * b_per_w
        writes = [None, None]
        for bi in range(b_per_w):
            s = bi % 2
            if writes[s] is not None:
                writes[s].wait()
            rp_row = rp_v[pl.ds((b0 + bi) * L, 16)]
            copies = [
                pltpu.async_copy(
                    p_hbm.at[rp_row[l]], buf.at[s, :, :, l], sem
                )
                for l in live_l
            ]
            for cp in copies:
                cp.wait()
            writes[s] = pltpu.async_copy(buf.at[s], out_hbm.at[b0 + bi], wsem)
        for wr in writes:
            wr.wait()

    return k(p4, rp_flat, zeros_blk)


def _tc_dense(x3, pz8, wk5, bk5, B, C, T, L, N, TT, live_l, bb=16):
    n_live = len(live_l)

    def body(x_ref, g_ref, w_ref, b_ref, o_ref):
        for s in range(bb):
            xc = (
                jnp.dot(
                    w_ref[...].astype(jnp.bfloat16),
                    x_ref[s].astype(jnp.bfloat16),
                    preferred_element_type=jnp.float32,
                )
                + b_ref[...]
            )
            gv = g_ref[s]
            for tt in range(TT):
                acc = None
                for i in range(n_live):
                    gtl = gv[:, tt, live_l[i], :]
                    xcl = xc[i * C:(i + 1) * C, tt * 128:(tt + 1) * 128]
                    term = gtl[:, None, :] * xcl[None, :, :]
                    acc = term if acc is None else acc + term
                o_ref[s, :, :, tt * 128:(tt + 1) * 128] = jnp.maximum(acc, 0.0)

    return pl.pallas_call(
        body,
        grid=(B // bb,),
        in_specs=[
            pl.BlockSpec((bb, C * L, T), lambda b: (b, 0, 0)),
            pl.BlockSpec((bb, N, TT, L, 128), lambda b: (b, 0, 0, 0, 0)),
            pl.BlockSpec((n_live * C, C * L), lambda b: (0, 0)),
            pl.BlockSpec((n_live * C, 1), lambda b: (0, 0)),
        ],
        out_specs=pl.BlockSpec((bb, N, C, T), lambda b: (b, 0, 0, 0)),
        out_shape=jax.ShapeDtypeStruct((B, N, C, T), jnp.float32),
        compiler_params=pltpu.CompilerParams(
            dimension_semantics=("arbitrary",),
        ),
    )(x3, pz8, wk5, bk5)


def kernel(x, P, remain_people, conv_w, conv_b):
    B, C, T, L = x.shape
    _, M, N = P.shape
    TT = T // 128
    live_l = tuple(l for l in range(L) if l not in _ZERO_L)
    n_live = len(live_l)

    x3 = jnp.swapaxes(x, 2, 3).reshape(B, C * L, T)
    p4 = P.transpose(1, 2, 0).reshape(M, N, TT, 128)
    rp_flat = remain_people.reshape(B * L).astype(jnp.int32)
    zeros_blk = jnp.zeros((N, TT, L, 128), jnp.float32)
    sel = jnp.zeros((n_live, L), jnp.float32).at[
        jnp.arange(n_live), jnp.array(live_l)
    ].set(1.0)
    wk5 = (sel[:, None, None, :] * conv_w[None, :, :, None]).reshape(
        n_live * C, C * L
    )
    bk5 = jnp.tile(conv_b, n_live).reshape(n_live * C, 1)

    pz8 = _sc_gather(p4, rp_flat, zeros_blk, B, T, L, M, N, TT)

    out7 = _tc_dense(x3, pz8, wk5, bk5, B, C, T, L, N, TT, live_l)

    out = jnp.transpose(out7, (0, 2, 3, 1))
    pz = jnp.transpose(pz8, (0, 2, 4, 3, 1)).reshape(B, T, L, N)
    return out, pz


# speedup vs baseline: 1.0321x; 1.0321x over previous
import functools

import jax
import jax.numpy as jnp
from jax import lax
from jax.experimental import pallas as pl
from jax.experimental.pallas import tpu as pltpu
from jax.experimental.pallas import tpu_sc as plsc

_ZERO_L = (1, 3, 4)


def _sc_gather(p4, rp_flat, zeros_blk, B, T, L, M, N, TT):
    NC, NS = 2, 16
    NW = NC * NS
    b_per_w = B // NW
    live_l = tuple(l for l in range(L) if l not in _ZERO_L)

    mesh = plsc.VectorSubcoreMesh(core_axis_name="c", subcore_axis_name="s")

    @functools.partial(
        pl.kernel,
        mesh=mesh,
        compiler_params=pltpu.CompilerParams(use_tc_tiling_on_sc=False),
        out_type=jax.ShapeDtypeStruct((B, N, TT, L, 128), jnp.float32),
        scratch_types=[
            pltpu.VMEM((N, TT, L, 128), jnp.float32),
            pltpu.VMEM((B * L + 8,), jnp.int32),
            pltpu.SemaphoreType.DMA,
        ],
    )
    def k(p_hbm, rp_hbm, z_hbm, out_hbm, buf, rp_v, sem):
        wid = lax.axis_index("s") * NC + lax.axis_index("c")
        pltpu.sync_copy(rp_hbm, rp_v.at[pl.ds(0, B * L)])
        pltpu.sync_copy(z_hbm, buf)

        def per_b(bi, carry):
            b = wid * b_per_w + bi
            rp_row = rp_v[pl.ds(b * L, 16)]
            copies = []
            for l in live_l:
                m = rp_row[l]
                for tt in range(TT):
                    copies.append(
                        pltpu.async_copy(
                            p_hbm.at[m, :, tt], buf.at[:, tt, l], sem
                        )
                    )
            for cp in copies:
                cp.wait()
            pltpu.sync_copy(buf, out_hbm.at[b])
            return carry

        lax.fori_loop(0, b_per_w, per_b, 0)

    return k(p4, rp_flat, zeros_blk)


def _tc_dense(x3, pz8, wk5, bk5, B, C, T, L, N, TT, live_l, bb=16):
    n_live = len(live_l)

    def body(x_ref, g_ref, w_ref, b_ref, o_ref):
        for s in range(bb):
            xc = (
                jnp.dot(
                    w_ref[...].astype(jnp.bfloat16),
                    x_ref[s].astype(jnp.bfloat16),
                    preferred_element_type=jnp.float32,
                )
                + b_ref[...]
            )
            gv = g_ref[s]
            for tt in range(TT):
                acc = None
                for i in range(n_live):
                    gtl = gv[:, tt, live_l[i], :]
                    xcl = xc[i * C:(i + 1) * C, tt * 128:(tt + 1) * 128]
                    term = gtl[:, None, :] * xcl[None, :, :]
                    acc = term if acc is None else acc + term
                o_ref[s, :, :, tt * 128:(tt + 1) * 128] = jnp.maximum(acc, 0.0)

    return pl.pallas_call(
        body,
        grid=(B // bb,),
        in_specs=[
            pl.BlockSpec((bb, C * L, T), lambda b: (b, 0, 0)),
            pl.BlockSpec((bb, N, TT, L, 128), lambda b: (b, 0, 0, 0, 0)),
            pl.BlockSpec((n_live * C, C * L), lambda b: (0, 0)),
            pl.BlockSpec((n_live * C, 1), lambda b: (0, 0)),
        ],
        out_specs=pl.BlockSpec((bb, N, C, T), lambda b: (b, 0, 0, 0)),
        out_shape=jax.ShapeDtypeStruct((B, N, C, T), jnp.float32),
        compiler_params=pltpu.CompilerParams(
            dimension_semantics=("arbitrary",),
        ),
    )(x3, pz8, wk5, bk5)


def kernel(x, P, remain_people, conv_w, conv_b):
    B, C, T, L = x.shape
    _, M, N = P.shape
    TT = T // 128
    live_l = tuple(l for l in range(L) if l not in _ZERO_L)
    n_live = len(live_l)

    x3 = jnp.swapaxes(x, 2, 3).reshape(B, C * L, T)
    p4 = P.transpose(1, 2, 0).reshape(M, N, TT, 128)
    rp_flat = remain_people.reshape(B * L).astype(jnp.int32)
    zeros_blk = jnp.zeros((N, TT, L, 128), jnp.float32)
    sel = jnp.zeros((n_live, L), jnp.float32).at[
        jnp.arange(n_live), jnp.array(live_l)
    ].set(1.0)
    wk5 = (sel[:, None, None, :] * conv_w[None, :, :, None]).reshape(
        n_live * C, C * L
    )
    bk5 = jnp.tile(conv_b, n_live).reshape(n_live * C, 1)

    pz8 = _sc_gather(p4, rp_flat, zeros_blk, B, T, L, M, N, TT)

    out7 = _tc_dense(x3, pz8, wk5, bk5, B, C, T, L, N, TT, live_l)

    out = jnp.transpose(out7, (0, 2, 3, 1))
    pz = jnp.transpose(pz8, (0, 2, 4, 3, 1)).reshape(B, T, L, N)
    return out, pz
